# Initial kernel scaffold; baseline (speedup 1.0000x reference)
import jax
import jax.numpy as jnp
from jax.experimental import pallas as pl


def kernel(input_ids, shp_tensor, special_embedding, full_embed, ln_w, ln_b, W1, b1, W2, b2):
    raise NotImplementedError("write your pallas kernel here")


# speedup vs baseline: 20.5039x; 20.5039x over previous
import functools

import jax
import jax.numpy as jnp
import numpy as np
from jax import lax
from jax.experimental import pallas as pl
from jax.experimental.pallas import tpu as pltpu

N_SPECIAL = 25
S = 20
K = 512
T = 512


def _gelu_exact(x):
    return 0.5 * x * (1.0 + lax.erf(x * np.float32(1.0 / np.sqrt(2.0))))


def _body(ids_ref, shpt_ref, g_ref, lnw_ref, lnb_ref, w1_ref, b1_ref,
          w2_ref, b2_ref, out_ref, gate_scr, *, seq_len, msl):
    i = pl.program_id(0)

    @pl.when(i == 0)
    def _table():
        gt = g_ref[...]
        mu = jnp.mean(gt, axis=1, keepdims=True)
        var = jnp.mean(gt * gt, axis=1, keepdims=True) - mu * mu
        xn = (gt - mu) * lax.rsqrt(var + 1e-5) * lnw_ref[...] + lnb_ref[...]
        h = lax.dot_general(xn, w1_ref[...], (((1,), (0,)), ((), ())),
                            preferred_element_type=jnp.float32) + b1_ref[...]
        h = _gelu_exact(h)
        logits = lax.dot_general(w2_ref[...], h, (((1,), (1,)), ((), ())),
                                 preferred_element_type=jnp.float32)
        gate_scr[...] = jax.nn.sigmoid(logits + b2_ref[...])

    ids = ids_ref[...]
    kcol = lax.broadcasted_iota(jnp.int32, (K, 1), 0)
    smask = kcol == ids
    sf = smask.astype(jnp.float32)
    gate = lax.dot_general(gate_scr[...], sf, (((1,), (0,)), ((), ())),
                           preferred_element_type=jnp.float32)
    spec = ids < N_SPECIAL
    jrow = (i * T + lax.broadcasted_iota(jnp.int32, (1, T), 1)) % seq_len
    regv = jnp.logical_and(jnp.logical_not(spec),
                           jnp.logical_and(jrow >= 1, jrow <= msl))
    specf = spec.astype(jnp.float32)
    regf = regv.astype(jnp.float32)
    ca = specf + regf * (1.0 - gate)
    cb = regf * gate
    qs = (kcol + (100 - N_SPECIAL)) // S
    seqs = (jnp.maximum(ids, N_SPECIAL) + (100 - N_SPECIAL)) // S
    mmask = qs == seqs
    w = ca * sf + jnp.where(mmask, cb * shpt_ref[...], 0.0)
    out_ref[...] = lax.dot_general(w, g_ref[...], (((0,), (0,)), ((), ())),
                                   preferred_element_type=jnp.float32)


def kernel(input_ids, shp_tensor, special_embedding, full_embed,
           ln_w, ln_b, W1, b1, W2, b2):
    B, L = input_ids.shape
    D = special_embedding.shape[1]
    n_tok = B * L
    nb = n_tok // T
    msl = min(shp_tensor.shape[1], L - 2)

    ids2 = input_ids.reshape(1, n_tok)
    shp_pad = jnp.zeros((B, L, S), shp_tensor.dtype)
    shp_pad = shp_pad.at[:, 1:1 + msl, :].set(shp_tensor[:, :msl])
    shpt = shp_pad.reshape(n_tok, S).T
    idxcols = np.mod(np.arange(K) - N_SPECIAL, S)
    shpt_full = jnp.take(shpt, idxcols, axis=0)
    g512 = jnp.concatenate([
        special_embedding[:N_SPECIAL],
        full_embed.reshape(-1, D),
        jnp.zeros((K - N_SPECIAL - full_embed.shape[0] * S, D),
                  full_embed.dtype)], axis=0)

    out = pl.pallas_call(
        functools.partial(_body, seq_len=L, msl=msl),
        grid=(nb,),
        in_specs=[
            pl.BlockSpec((1, T), lambda i: (0, i)),
            pl.BlockSpec((K, T), lambda i: (0, i)),
            pl.BlockSpec((K, D), lambda i: (0, 0)),
            pl.BlockSpec((1, D), lambda i: (0, 0)),
            pl.BlockSpec((1, D), lambda i: (0, 0)),
            pl.BlockSpec((D, D), lambda i: (0, 0)),
            pl.BlockSpec((1, D), lambda i: (0, 0)),
            pl.BlockSpec((1, D), lambda i: (0, 0)),
            pl.BlockSpec((1, 1), lambda i: (0, 0)),
        ],
        out_specs=pl.BlockSpec((T, D), lambda i: (i, 0)),
        out_shape=jax.ShapeDtypeStruct((n_tok, D), jnp.float32),
        scratch_shapes=[pltpu.VMEM((1, K), jnp.float32)],
    )(ids2, shpt_full, g512, ln_w.reshape(1, D), ln_b.reshape(1, D),
      W1, b1.reshape(1, D), W2.reshape(1, D), b2.reshape(1, 1))
    return out.reshape(B, L, D)


# speedup vs baseline: 22.3468x; 1.0899x over previous
import functools

import jax
import jax.numpy as jnp
import numpy as np
from jax import lax
from jax.experimental import pallas as pl
from jax.experimental.pallas import tpu as pltpu

N_SPECIAL = 25
S = 20
K = 512
T = 512


def _gelu_exact(x):
    return 0.5 * x * (1.0 + lax.erf(x * np.float32(1.0 / np.sqrt(2.0))))


def _body(ids_ref, shp_ref, g_ref, lnw_ref, lnb_ref, w1_ref, b1_ref,
          w2_ref, b2_ref, out_ref, gate_scr, *, seq_len, msl):
    i = pl.program_id(0)

    @pl.when(i == 0)
    def _table():
        gt = g_ref[...].astype(jnp.float32)
        mu = jnp.mean(gt, axis=1, keepdims=True)
        var = jnp.mean(gt * gt, axis=1, keepdims=True) - mu * mu
        xn = (gt - mu) * lax.rsqrt(var + 1e-5) * lnw_ref[...] + lnb_ref[...]
        h = lax.dot_general(xn.astype(jnp.bfloat16), w1_ref[...],
                            (((1,), (0,)), ((), ())),
                            preferred_element_type=jnp.float32) + b1_ref[...]
        h = _gelu_exact(h)
        logits = lax.dot_general(w2_ref[...], h, (((1,), (1,)), ((), ())),
                                 preferred_element_type=jnp.float32)
        gate_scr[...] = jax.nn.sigmoid(logits + b2_ref[...])

    ids = ids_ref[...]
    kcol = lax.broadcasted_iota(jnp.int32, (K, 1), 0)
    smask = kcol == ids
    gate = lax.dot_general(gate_scr[...].astype(jnp.bfloat16),
                           smask.astype(jnp.bfloat16),
                           (((1,), (0,)), ((), ())),
                           preferred_element_type=jnp.float32)
    spec = ids < N_SPECIAL
    jrow = (i * T + lax.broadcasted_iota(jnp.int32, (1, T), 1)) % seq_len
    regv = jnp.logical_and(jnp.logical_not(spec),
                           jnp.logical_and(jrow >= 1, jrow <= msl))
    ca = jnp.where(spec, 1.0, 0.0) + regv.astype(jnp.float32) * (1.0 - gate)
    cb = regv.astype(jnp.float32) * gate
    srow = lax.broadcasted_iota(jnp.int32, (1, S), 1)
    rrep = ((kcol + (S - N_SPECIAL % S)) % S == srow).astype(jnp.float32)
    shpt = lax.dot_general(rrep, shp_ref[...], (((1,), (1,)), ((), ())),
                           preferred_element_type=jnp.float32)
    qs = (kcol + (100 - N_SPECIAL)) // S
    seqs = (jnp.maximum(ids, N_SPECIAL) + (100 - N_SPECIAL)) // S
    mmask = qs == seqs
    w = jnp.where(smask, ca, 0.0) + jnp.where(mmask, cb * shpt, 0.0)
    out_ref[...] = lax.dot_general(w.astype(jnp.bfloat16), g_ref[...],
                                   (((0,), (0,)), ((), ())),
                                   preferred_element_type=jnp.float32)


def kernel(input_ids, shp_tensor, special_embedding, full_embed,
           ln_w, ln_b, W1, b1, W2, b2):
    B, L = input_ids.shape
    D = special_embedding.shape[1]
    n_tok = B * L
    nb = n_tok // T
    msl = min(shp_tensor.shape[1], L - 2)

    ids2 = input_ids.reshape(1, n_tok)
    shp_pad = jnp.pad(shp_tensor, ((0, 0), (1, L - 1 - msl), (0, 0)))
    shp_pad = shp_pad.reshape(n_tok, S)
    g512 = jnp.concatenate([
        special_embedding[:N_SPECIAL],
        full_embed.reshape(-1, D),
        jnp.zeros((K - N_SPECIAL - full_embed.shape[0] * S, D),
                  full_embed.dtype)], axis=0).astype(jnp.bfloat16)

    out = pl.pallas_call(
        functools.partial(_body, seq_len=L, msl=msl),
        grid=(nb,),
        in_specs=[
            pl.BlockSpec((1, T), lambda i: (0, i)),
            pl.BlockSpec((T, S), lambda i: (i, 0)),
            pl.BlockSpec((K, D), lambda i: (0, 0)),
            pl.BlockSpec((1, D), lambda i: (0, 0)),
            pl.BlockSpec((1, D), lambda i: (0, 0)),
            pl.BlockSpec((D, D), lambda i: (0, 0)),
            pl.BlockSpec((1, D), lambda i: (0, 0)),
            pl.BlockSpec((1, D), lambda i: (0, 0)),
            pl.BlockSpec((1, 1), lambda i: (0, 0)),
        ],
        out_specs=pl.BlockSpec((T, D), lambda i: (i, 0)),
        out_shape=jax.ShapeDtypeStruct((n_tok, D), jnp.float32),
        scratch_shapes=[pltpu.VMEM((1, K), jnp.float32)],
    )(ids2, shp_pad, g512, ln_w.reshape(1, D), ln_b.reshape(1, D),
      W1.astype(jnp.bfloat16), b1.reshape(1, D), W2.reshape(1, D),
      b2.reshape(1, 1))
    return out.reshape(B, L, D)


# speedup vs baseline: 26.4455x; 1.1834x over previous
import functools

import jax
import jax.numpy as jnp
import numpy as np
from jax import lax
from jax.experimental import pallas as pl
from jax.experimental.pallas import tpu as pltpu

N_SPECIAL = 25
S = 20
KR = 400
T = 512


def _gelu_exact(x):
    return 0.5 * x * (1.0 + lax.erf(x * np.float32(1.0 / np.sqrt(2.0))))


def _body(ids_ref, shp_ref, sp_ref, fe_ref, lnw_ref, lnb_ref, w1_ref,
          b1_ref, w2_ref, b2_ref, out_ref, gate_scr, feb_scr,
          *, seq_len, msl):
    i = pl.program_id(0)

    @pl.when(i == 0)
    def _table():
        fe = fe_ref[...]
        mu = jnp.mean(fe, axis=1, keepdims=True)
        var = jnp.mean(fe * fe, axis=1, keepdims=True) - mu * mu
        xn = (fe - mu) * lax.rsqrt(var + 1e-5) * lnw_ref[...] + lnb_ref[...]
        h = lax.dot_general(xn.astype(jnp.bfloat16),
                            w1_ref[...].astype(jnp.bfloat16),
                            (((1,), (0,)), ((), ())),
                            preferred_element_type=jnp.float32) + b1_ref[...]
        h = _gelu_exact(h)
        logits = lax.dot_general(w2_ref[...], h, (((1,), (1,)), ((), ())),
                                 preferred_element_type=jnp.float32)
        gate_scr[...] = jax.nn.sigmoid(logits + b2_ref[...])
        feb_scr[...] = fe.astype(jnp.bfloat16)

    ids = ids_ref[...]
    idr = ids - N_SPECIAL
    kcol = lax.broadcasted_iota(jnp.int32, (KR, 1), 0)
    smask = kcol == idr
    sb = smask.astype(jnp.bfloat16)
    gate = lax.dot_general(gate_scr[...].astype(jnp.bfloat16), sb,
                           (((1,), (0,)), ((), ())),
                           preferred_element_type=jnp.float32)
    spec = ids < N_SPECIAL
    jrow = (i * T + lax.broadcasted_iota(jnp.int32, (1, T), 1)) % seq_len
    regv = jnp.logical_and(jnp.logical_not(spec),
                           jnp.logical_and(jrow >= 1, jrow <= msl))
    regf = regv.astype(jnp.float32)
    ca = regf * (1.0 - gate)
    cb = regf * gate
    srow = lax.broadcasted_iota(jnp.int32, (1, S), 1)
    rrep = (kcol % S == srow).astype(jnp.float32)
    shpt = lax.dot_general(rrep, shp_ref[...], (((1,), (1,)), ((), ())),
                           preferred_element_type=jnp.float32)
    qs = kcol // S
    seqs = jnp.maximum(idr, 0) // S
    mmask = qs == seqs
    wreg = jnp.where(smask, ca, 0.0) + jnp.where(mmask, cb * shpt, 0.0)
    k26 = lax.broadcasted_iota(jnp.int32, (sp_ref.shape[0], 1), 0)
    wsp = jnp.logical_and(k26 == ids, spec).astype(jnp.bfloat16)
    out_ref[...] = (
        lax.dot_general(wreg.astype(jnp.bfloat16), feb_scr[...],
                        (((0,), (0,)), ((), ())),
                        preferred_element_type=jnp.float32)
        + lax.dot_general(wsp, sp_ref[...].astype(jnp.bfloat16),
                          (((0,), (0,)), ((), ())),
                          preferred_element_type=jnp.float32))


def kernel(input_ids, shp_tensor, special_embedding, full_embed,
           ln_w, ln_b, W1, b1, W2, b2):
    B, L = input_ids.shape
    D = special_embedding.shape[1]
    n_tok = B * L
    nb = n_tok // T
    msl = min(shp_tensor.shape[1], L - 2)
    nsp = special_embedding.shape[0]

    ids2 = input_ids.reshape(1, n_tok)
    shp_pad = jnp.pad(shp_tensor, ((0, 0), (1, L - 1 - msl), (0, 0)))
    shp_pad = shp_pad.reshape(n_tok, S)
    fe = full_embed.reshape(KR, D)

    out = pl.pallas_call(
        functools.partial(_body, seq_len=L, msl=msl),
        grid=(nb,),
        in_specs=[
            pl.BlockSpec((1, T), lambda i: (0, i)),
            pl.BlockSpec((T, S), lambda i: (i, 0)),
            pl.BlockSpec((nsp, D), lambda i: (0, 0)),
            pl.BlockSpec((KR, D), lambda i: (0, 0)),
            pl.BlockSpec((1, D), lambda i: (0, 0)),
            pl.BlockSpec((1, D), lambda i: (0, 0)),
            pl.BlockSpec((D, D), lambda i: (0, 0)),
            pl.BlockSpec((1, D), lambda i: (0, 0)),
            pl.BlockSpec((1, D), lambda i: (0, 0)),
            pl.BlockSpec((1, 1), lambda i: (0, 0)),
        ],
        out_specs=pl.BlockSpec((T, D), lambda i: (i, 0)),
        out_shape=jax.ShapeDtypeStruct((n_tok, D), jnp.float32),
        scratch_shapes=[pltpu.VMEM((1, KR), jnp.float32),
                        pltpu.VMEM((KR, D), jnp.bfloat16)],
    )(ids2, shp_pad, special_embedding, fe, ln_w.reshape(1, D),
      ln_b.reshape(1, D), W1, b1.reshape(1, D), W2.reshape(1, D),
      b2.reshape(1, 1))
    return out.reshape(B, L, D)
